# baseline (speedup 1.0000x reference)
import functools

import jax
import jax.numpy as jnp
from jax.experimental import pallas as pl
from jax.experimental.pallas import tpu as pltpu

ROWS_PER_STEP = 8


def _body(idx_ref, tgt_ref, *refs, vocab):
    row_refs = refs[:ROWS_PER_STEP]
    out_ref, loss_ref = refs[ROWS_PER_STEP], refs[ROWS_PER_STEP + 1]
    i = pl.program_id(0)

    @pl.when(i == 0)
    def _init():
        loss_ref[0, 0] = 0.0

    col = jax.lax.broadcasted_iota(jnp.int32, (1, vocab), 1)
    s = jnp.float32(0.0)
    for j, row_ref in enumerate(row_refs):
        row = row_ref[0]
        out_ref[pl.ds(j, 1), :] = row
        m = jnp.max(row)
        lse = jnp.log(jnp.sum(jnp.exp(row - m))) + m
        tgt = tgt_ref[i * ROWS_PER_STEP + j]
        tgt_logit = jnp.sum(jnp.where(col == tgt, row, 0.0))
        s += lse - tgt_logit
    loss_ref[0, 0] += s

    @pl.when(i == pl.num_programs(0) - 1)
    def _fin():
        loss_ref[0, 0] = loss_ref[0, 0] / (pl.num_programs(0) * ROWS_PER_STEP)


def kernel(indices, targets, table):
    B, T = indices.shape
    vocab = table.shape[1]
    n = B * T
    flat_idx = indices.reshape(n).astype(jnp.int32)
    flat_tgt = targets.reshape(n).astype(jnp.int32)

    grid = n // ROWS_PER_STEP

    def row_map(j):
        return lambda i, idx_ref, tgt_ref: (idx_ref[i * ROWS_PER_STEP + j], 0, 0)

    in_specs = [
        pl.BlockSpec((1, 1, vocab), row_map(j)) for j in range(ROWS_PER_STEP)
    ]
    out_specs = [
        pl.BlockSpec((ROWS_PER_STEP, vocab), lambda i, idx_ref, tgt_ref: (i, 0)),
        pl.BlockSpec((1, 1), lambda i, idx_ref, tgt_ref: (0, 0),
                     memory_space=pltpu.SMEM),
    ]
    grid_spec = pltpu.PrefetchScalarGridSpec(
        num_scalar_prefetch=2,
        grid=(grid,),
        in_specs=in_specs,
        out_specs=out_specs,
    )
    logits, loss = pl.pallas_call(
        functools.partial(_body, vocab=vocab),
        grid_spec=grid_spec,
        out_shape=[
            jax.ShapeDtypeStruct((n, vocab), jnp.float32),
            jax.ShapeDtypeStruct((1, 1), jnp.float32),
        ],
    )(flat_idx, flat_tgt, *([table.reshape(vocab, 1, vocab)] * ROWS_PER_STEP))
    return logits.reshape(B, T, vocab), loss[0, 0]


# speedup vs baseline: 3.7297x; 3.7297x over previous
import functools

import jax
import jax.numpy as jnp
from jax.experimental import pallas as pl
from jax.experimental.pallas import tpu as pltpu

ROWS_PER_STEP = 32


def _body(idx_ref, tgt_ref, *refs, vocab):
    row_refs = refs[:ROWS_PER_STEP]
    out_ref, loss_ref = refs[ROWS_PER_STEP], refs[ROWS_PER_STEP + 1]
    i = pl.program_id(0)

    @pl.when(i == 0)
    def _init():
        loss_ref[0, 0] = 0.0

    for j, row_ref in enumerate(row_refs):
        out_ref[pl.ds(j, 1), :] = row_ref[0]

    blk = out_ref[...]
    m = jnp.max(blk, axis=1, keepdims=True)
    lse = jnp.log(jnp.sum(jnp.exp(blk - m), axis=1, keepdims=True)) + m
    tgts = jnp.stack(
        [tgt_ref[i * ROWS_PER_STEP + j] for j in range(ROWS_PER_STEP)]
    ).reshape(ROWS_PER_STEP, 1)
    col = jax.lax.broadcasted_iota(jnp.int32, (ROWS_PER_STEP, vocab), 1)
    tgt_logit = jnp.sum(
        jnp.where(col == tgts, blk, 0.0), axis=1, keepdims=True
    )
    loss_ref[0, 0] += jnp.sum(lse - tgt_logit)

    @pl.when(i == pl.num_programs(0) - 1)
    def _fin():
        loss_ref[0, 0] = loss_ref[0, 0] / (pl.num_programs(0) * ROWS_PER_STEP)


def kernel(indices, targets, table):
    B, T = indices.shape
    vocab = table.shape[1]
    n = B * T
    flat_idx = indices.reshape(n).astype(jnp.int32)
    flat_tgt = targets.reshape(n).astype(jnp.int32)

    grid = n // ROWS_PER_STEP

    def row_map(j):
        return lambda i, idx_ref, tgt_ref: (idx_ref[i * ROWS_PER_STEP + j], 0, 0)

    in_specs = [
        pl.BlockSpec((1, 1, vocab), row_map(j)) for j in range(ROWS_PER_STEP)
    ]
    out_specs = [
        pl.BlockSpec((ROWS_PER_STEP, vocab), lambda i, idx_ref, tgt_ref: (i, 0)),
        pl.BlockSpec((1, 1), lambda i, idx_ref, tgt_ref: (0, 0),
                     memory_space=pltpu.SMEM),
    ]
    grid_spec = pltpu.PrefetchScalarGridSpec(
        num_scalar_prefetch=2,
        grid=(grid,),
        in_specs=in_specs,
        out_specs=out_specs,
    )
    logits, loss = pl.pallas_call(
        functools.partial(_body, vocab=vocab),
        grid_spec=grid_spec,
        out_shape=[
            jax.ShapeDtypeStruct((n, vocab), jnp.float32),
            jax.ShapeDtypeStruct((1, 1), jnp.float32),
        ],
    )(flat_idx, flat_tgt, *([table.reshape(vocab, 1, vocab)] * ROWS_PER_STEP))
    return logits.reshape(B, T, vocab), loss[0, 0]
